# baseline (speedup 1.0000x reference)
import functools

import jax
import jax.numpy as jnp
from jax import lax
from jax.experimental import pallas as pl
from jax.experimental.pallas import tpu as pltpu
from jax.experimental.pallas import tpu_sc as plsc

DEPTH = 4
H = 384
HW = H // 2
HP = 256
HB = H // 32


def _lo(v):
    return lax.bitcast_convert_type(v << 16, jnp.float32)


def _hi(v):
    return lax.bitcast_convert_type(v & jnp.int32(-65536), jnp.float32)


def _pack(e, o):
    ei = lax.bitcast_convert_type(e, jnp.int32)
    oi = lax.bitcast_convert_type(o, jnp.int32)
    ei = ei + jnp.int32(0x7FFF) + ((ei >> 16) & jnp.int32(1))
    oi = oi + jnp.int32(0x7FFF) + ((oi >> 16) & jnp.int32(1))
    return lax.shift_right_logical(ei, 16) | (oi & jnp.int32(-65536))


def _mm_first(f_bonds, W_i, W_h, blk=1280):
    nb, bd = f_bonds.shape

    def body(fb_ref, wi_ref, wh_ref, inp_ref, mh_ref):
        inp = jnp.dot(fb_ref[...], wi_ref[...], preferred_element_type=jnp.float32)
        inp_ref[...] = inp.astype(jnp.bfloat16)
        mh = jnp.dot(jnp.maximum(inp, 0.0), wh_ref[...],
                     preferred_element_type=jnp.float32)
        mh_ref[:, :H] = mh.astype(jnp.bfloat16)
        mh_ref[:, H:] = jnp.zeros((blk, 2 * HP - H), jnp.bfloat16)

    return pl.pallas_call(
        body,
        grid=(nb // blk,),
        in_specs=[
            pl.BlockSpec((blk, bd), lambda i: (i, 0)),
            pl.BlockSpec((bd, H), lambda i: (0, 0)),
            pl.BlockSpec((H, H), lambda i: (0, 0)),
        ],
        out_specs=[
            pl.BlockSpec((blk, H), lambda i: (i, 0)),
            pl.BlockSpec((blk, 2 * HP), lambda i: (i, 0)),
        ],
        out_shape=[
            jax.ShapeDtypeStruct((nb, H), jnp.bfloat16),
            jax.ShapeDtypeStruct((nb, 2 * HP), jnp.bfloat16),
        ],
    )(f_bonds, W_i, W_h)


def _mm_matmul(x, W, blk=1280):
    n = x.shape[0]

    def body(x_ref, w_ref, out_ref):
        xf = x_ref[:, :H].astype(jnp.float32)
        y = jnp.dot(xf, w_ref[...], preferred_element_type=jnp.float32)
        out_ref[:, :H] = y.astype(jnp.bfloat16)
        out_ref[:, H:] = jnp.zeros((blk, 2 * HP - H), jnp.bfloat16)

    return pl.pallas_call(
        body,
        grid=(n // blk,),
        in_specs=[
            pl.BlockSpec((blk, 2 * HP), lambda i: (i, 0)),
            pl.BlockSpec((H, H), lambda i: (0, 0)),
        ],
        out_specs=pl.BlockSpec((blk, 2 * HP), lambda i: (i, 0)),
        out_shape=jax.ShapeDtypeStruct((n, 2 * HP), jnp.bfloat16),
    )(x, W)


def _mm_out1(f_atoms, W_o1, b_o, blk=2000):
    na, fd = f_atoms.shape
    b2d = b_o.reshape(1, H)

    def body(fa_ref, w1_ref, b_ref, out_ref):
        out_ref[...] = jnp.dot(fa_ref[...], w1_ref[...],
                               preferred_element_type=jnp.float32) + b_ref[...]

    return pl.pallas_call(
        body,
        grid=(na // blk,),
        in_specs=[
            pl.BlockSpec((blk, fd), lambda i: (i, 0)),
            pl.BlockSpec((fd, H), lambda i: (0, 0)),
            pl.BlockSpec((1, H), lambda i: (0, 0)),
        ],
        out_specs=pl.BlockSpec((blk, H), lambda i: (i, 0)),
        out_shape=jax.ShapeDtypeStruct((na, H), jnp.float32),
    )(f_atoms, W_o1, b2d)


def _mm_out2(P, a_msg, W_o2, blk=2000):
    na = P.shape[0]

    def body(p_ref, am_ref, w2_ref, out_ref):
        am = am_ref[:, :H].astype(jnp.float32)
        acc = p_ref[...] + jnp.dot(am, w2_ref[...],
                                   preferred_element_type=jnp.float32)
        out_ref[...] = jnp.maximum(acc, 0.0)

    return pl.pallas_call(
        body,
        grid=(na // blk,),
        in_specs=[
            pl.BlockSpec((blk, H), lambda i: (i, 0)),
            pl.BlockSpec((blk, 2 * HP), lambda i: (i, 0)),
            pl.BlockSpec((H, H), lambda i: (0, 0)),
        ],
        out_specs=pl.BlockSpec((blk, H), lambda i: (i, 0)),
        out_shape=jax.ShapeDtypeStruct((na, H), jnp.float32),
    )(P, a_msg, W_o2)


def _gather_sum(table, a2bT_flat, n_pad, max_nb, nc, ns, ac=80):
    nw = nc * ns
    cs = n_pad // (nw * ac)
    mesh = plsc.VectorSubcoreMesh(core_axis_name="c", subcore_axis_name="s")

    @functools.partial(
        pl.kernel,
        mesh=mesh,
        out_type=jax.ShapeDtypeStruct((n_pad, HP), jnp.int32),
        scratch_types=[
            pltpu.VMEM((ac,), jnp.int32),
            pltpu.VMEM((ac,), jnp.int32),
            pltpu.VMEM((ac, H), jnp.float32),
            pltpu.VMEM((ac, HP), jnp.int32),
            pltpu.VMEM((ac, HP), jnp.int32),
            pltpu.VMEM((ac, HP), jnp.int32),
            pltpu.SemaphoreType.DMA,
            pltpu.SemaphoreType.DMA,
            pltpu.SemaphoreType.DMA,
        ],
    )
    def k(table_hbm, a2bT_hbm, out_hbm, idx_0, idx_1, acc_v, rows_0,
          rows_1, out_v, sem_0, sem_1, sem_o):
        c = lax.axis_index("c")
        s = lax.axis_index("s")
        w = c * ns + s
        idx = (idx_0, idx_1)
        rows = (rows_0, rows_1)
        sems = (sem_0, sem_1)

        def chunk(kk, _):
            atom_base = (w * cs + kk) * ac

            for g in (0, 1):
                off = g * n_pad + atom_base
                pltpu.sync_copy(a2bT_hbm.at[pl.ds(off, ac)], idx[g])
                pltpu.async_copy(table_hbm.at[idx[g]], rows[g], sems[g])

            for g in range(max_nb):
                b = g % 2
                pltpu.make_async_copy(table_hbm.at[idx[b]], rows[b],
                                      sems[b]).wait()
                rbuf = rows[b]

                if g == 0:
                    def row0(r, _):
                        for d in range(HB):
                            v = rbuf[r, pl.ds(d * 16, 16)]
                            acc_v[r, pl.ds(d * 32, 16)] = _lo(v)
                            acc_v[r, pl.ds(d * 32 + 16, 16)] = _hi(v)
                        return 0

                    lax.fori_loop(0, ac, row0, 0)
                else:
                    def rowa(r, _):
                        for d in range(HB):
                            v = rbuf[r, pl.ds(d * 16, 16)]
                            se = pl.ds(d * 32, 16)
                            so = pl.ds(d * 32 + 16, 16)
                            acc_v[r, se] = acc_v[r, se] + _lo(v)
                            acc_v[r, so] = acc_v[r, so] + _hi(v)
                        return 0

                    lax.fori_loop(0, ac, rowa, 0)

                if g + 2 < max_nb:
                    off = (g + 2) * n_pad + atom_base
                    pltpu.sync_copy(a2bT_hbm.at[pl.ds(off, ac)], idx[b])
                    pltpu.async_copy(table_hbm.at[idx[b]], rows[b], sems[b])

            @pl.when(kk > 0)
            def _():
                pltpu.make_async_copy(out_v, out_hbm.at[pl.ds(0, ac)],
                                      sem_o).wait()

            def rowp(r, _):
                for d in range(HB):
                    e = acc_v[r, pl.ds(d * 32, 16)]
                    o = acc_v[r, pl.ds(d * 32 + 16, 16)]
                    out_v[r, pl.ds(d * 16, 16)] = _pack(e, o)
                return 0

            lax.fori_loop(0, ac, rowp, 0)
            pltpu.async_copy(out_v, out_hbm.at[pl.ds(atom_base, ac)], sem_o)
            return 0

        lax.fori_loop(0, cs, chunk, 0)
        pltpu.make_async_copy(out_v, out_hbm.at[pl.ds(0, ac)], sem_o).wait()

    return k(table, a2bT_flat)


def _gather_msg(inp, mh, amh, b2a, b2revb, nc, ns, cr=40):
    nb = inp.shape[0]
    nw = nc * ns
    pw = nb // nw
    npair = pw // (2 * cr)
    mesh = plsc.VectorSubcoreMesh(core_axis_name="c", subcore_axis_name="s")

    @functools.partial(
        pl.kernel,
        mesh=mesh,
        out_type=jax.ShapeDtypeStruct((nb, HP), jnp.int32),
        scratch_types=[
            pltpu.VMEM((cr,), jnp.int32),
            pltpu.VMEM((cr,), jnp.int32),
            pltpu.VMEM((cr,), jnp.int32),
            pltpu.VMEM((cr,), jnp.int32),
            pltpu.VMEM((cr, HW), jnp.int32),
            pltpu.VMEM((cr, HW), jnp.int32),
            pltpu.VMEM((cr, HP), jnp.int32),
            pltpu.VMEM((cr, HP), jnp.int32),
            pltpu.VMEM((cr, HP), jnp.int32),
            pltpu.VMEM((cr, HP), jnp.int32),
            pltpu.VMEM((cr, HP), jnp.int32),
            pltpu.VMEM((cr, HP), jnp.int32),
            pltpu.SemaphoreType.DMA,
            pltpu.SemaphoreType.DMA,
            pltpu.SemaphoreType.DMA,
            pltpu.SemaphoreType.DMA,
        ],
    )
    def k(inp_hbm, mh_hbm, amh_hbm, b2a_hbm, b2revb_hbm, out_hbm,
          i1_0, i1_1, i2_0, i2_1, bi_0, bi_1, ba_0, ba_1, bb_0, bb_1,
          bo_0, bo_1, semi_0, semi_1, semo_0, semo_1):
        c = lax.axis_index("c")
        s = lax.axis_index("s")
        w = c * ns + s
        i1 = (i1_0, i1_1)
        i2 = (i2_0, i2_1)
        bi = (bi_0, bi_1)
        ba = (ba_0, ba_1)
        bb = (bb_0, bb_1)
        bo = (bo_0, bo_1)
        semi = (semi_0, semi_1)
        semo = (semo_0, semo_1)

        def issue_in(cc, sl):
            base = pl.multiple_of(w * pw + cc * cr, 8)
            pltpu.sync_copy(b2a_hbm.at[pl.ds(base, cr)], i1[sl])
            pltpu.sync_copy(b2revb_hbm.at[pl.ds(base, cr)], i2[sl])
            pltpu.async_copy(inp_hbm.at[pl.ds(base, cr)], bi[sl], semi[sl])
            pltpu.async_copy(amh_hbm.at[i1[sl]], ba[sl], semi[sl])
            pltpu.async_copy(mh_hbm.at[i2[sl]], bb[sl], semi[sl])

        def wait_in(sl):
            pltpu.make_async_copy(inp_hbm.at[pl.ds(0, cr)], bi[sl],
                                  semi[sl]).wait()
            pltpu.make_async_copy(amh_hbm.at[pl.ds(0, cr)], ba[sl],
                                  semi[sl]).wait()
            pltpu.make_async_copy(mh_hbm.at[pl.ds(0, cr)], bb[sl],
                                  semi[sl]).wait()

        def vpu(sl):
            bis, bas, bbs, bos = bi[sl], ba[sl], bb[sl], bo[sl]

            def row(r, _):
                for d in range(HB):
                    vi = bis[r, pl.ds(d * 16, 16)]
                    va = bas[r, pl.ds(d * 16, 16)]
                    vb = bbs[r, pl.ds(d * 16, 16)]
                    ve = jnp.maximum(_lo(vi) + _lo(va) - _lo(vb), 0.0)
                    vo = jnp.maximum(_hi(vi) + _hi(va) - _hi(vb), 0.0)
                    bos[r, pl.ds(d * 16, 16)] = _pack(ve, vo)
                return 0

            lax.fori_loop(0, cr, row, 0)

        def issue_out(cc, sl):
            base = pl.multiple_of(w * pw + cc * cr, 8)
            pltpu.async_copy(bo[sl], out_hbm.at[pl.ds(base, cr)], semo[sl])

        def wait_out(sl):
            pltpu.make_async_copy(bo[sl], out_hbm.at[pl.ds(0, cr)],
                                  semo[sl]).wait()

        issue_in(0, 0)

        def body(kk, _):
            c0 = 2 * kk
            c1 = 2 * kk + 1

            issue_in(c1, 1)
            wait_in(0)

            @pl.when(kk > 0)
            def _():
                wait_out(0)

            vpu(0)
            issue_out(c0, 0)
            wait_in(1)

            @pl.when(kk + 1 < npair)
            def _():
                issue_in(c0 + 2, 0)

            @pl.when(kk > 0)
            def _():
                wait_out(1)

            vpu(1)
            issue_out(c1, 1)
            return 0

        lax.fori_loop(0, npair, body, 0)
        wait_out(0)
        wait_out(1)

    return k(inp, mh, amh, b2a, b2revb)


def _as_i32(x_bf16):
    n, m = x_bf16.shape
    return lax.bitcast_convert_type(
        x_bf16.reshape(n, m // 2, 2), jnp.int32)


def _as_bf16(x_i32):
    n, m = x_i32.shape
    return lax.bitcast_convert_type(x_i32, jnp.bfloat16).reshape(n, 2 * m)


def kernel(f_atoms, f_bonds, a2b, b2a, b2revb, W_i, W_h, W_o, b_o):
    n_atoms, max_nb = a2b.shape
    fd = f_atoms.shape[1]

    info = plsc.get_sparse_core_info()
    nc, ns = info.num_cores, info.num_subcores
    nw = nc * ns
    ac = 80

    grp = nw * ac
    n_pad = ((n_atoms + grp - 1) // grp) * grp

    b2a = b2a.astype(jnp.int32)
    b2revb = b2revb.astype(jnp.int32)
    a2bT_flat = jnp.pad(a2b.astype(jnp.int32),
                        ((0, n_pad - n_atoms), (0, 0))).T.reshape(-1)

    P = _mm_out1(f_atoms, W_o[:fd], b_o)
    inp, mh = _mm_first(f_bonds, W_i, W_h)
    inp_i = _as_i32(inp)
    mh_i = _as_i32(mh)
    amh_i = _gather_sum(mh_i, a2bT_flat, n_pad, max_nb, nc, ns, ac)
    msg_i = _gather_msg(inp_i, mh_i, amh_i, b2a, b2revb, nc, ns)
    for _ in range(DEPTH - 2):
        mh_i = _as_i32(_mm_matmul(_as_bf16(msg_i), W_h))
        asum_i = _gather_sum(msg_i, a2bT_flat, n_pad, max_nb, nc, ns, ac)
        amh_i = _as_i32(_mm_matmul(_as_bf16(asum_i), W_h))
        msg_i = _gather_msg(inp_i, mh_i, amh_i, b2a, b2revb, nc, ns)

    a_msg_i = _gather_sum(msg_i, a2bT_flat, n_pad, max_nb, nc, ns, ac)
    return _mm_out2(P, _as_bf16(a_msg_i)[:n_atoms], W_o[fd:])


# speedup vs baseline: 3.5983x; 3.5983x over previous
import functools

import jax
import jax.numpy as jnp
from jax import lax
from jax.experimental import pallas as pl
from jax.experimental.pallas import tpu as pltpu
from jax.experimental.pallas import tpu_sc as plsc

DEPTH = 4
H = 384
HW = H // 2
HP = 256
HB = H // 32


def _lo(v):
    return lax.bitcast_convert_type(v << 16, jnp.float32)


def _hi(v):
    return lax.bitcast_convert_type(v & jnp.int32(-65536), jnp.float32)


def _pack(e, o):
    ei = lax.bitcast_convert_type(e, jnp.int32)
    oi = lax.bitcast_convert_type(o, jnp.int32)
    ei = ei + jnp.int32(0x7FFF) + ((ei >> 16) & jnp.int32(1))
    oi = oi + jnp.int32(0x7FFF) + ((oi >> 16) & jnp.int32(1))
    return lax.shift_right_logical(ei, 16) | (oi & jnp.int32(-65536))


def _unpack_tc(v):
    lo = lax.bitcast_convert_type(v << 16, jnp.float32)
    hi = lax.bitcast_convert_type(v & jnp.int32(-65536), jnp.float32)
    return jnp.concatenate([lo, hi], axis=1)


def _pack_tc(y):
    return _pack(y[:, :HW], y[:, HW:])


def _mm_first(f_bonds, W_i, W_h, blk=1280):
    nb, bd = f_bonds.shape

    def body(fb_ref, wi_ref, wh_ref, inp_ref, mh_ref):
        inp = jnp.dot(fb_ref[...], wi_ref[...], preferred_element_type=jnp.float32)
        inp_ref[...] = _pack_tc(inp)
        mh = jnp.dot(jnp.maximum(inp, 0.0), wh_ref[...],
                     preferred_element_type=jnp.float32)
        mh_ref[:, :HW] = _pack_tc(mh)
        mh_ref[:, HW:] = jnp.zeros((blk, HP - HW), jnp.int32)

    return pl.pallas_call(
        body,
        grid=(nb // blk,),
        in_specs=[
            pl.BlockSpec((blk, bd), lambda i: (i, 0)),
            pl.BlockSpec((bd, H), lambda i: (0, 0)),
            pl.BlockSpec((H, H), lambda i: (0, 0)),
        ],
        out_specs=[
            pl.BlockSpec((blk, HW), lambda i: (i, 0)),
            pl.BlockSpec((blk, HP), lambda i: (i, 0)),
        ],
        out_shape=[
            jax.ShapeDtypeStruct((nb, HW), jnp.int32),
            jax.ShapeDtypeStruct((nb, HP), jnp.int32),
        ],
    )(f_bonds, W_i, W_h)


def _mm_matmul(x, W, blk=1280):
    n = x.shape[0]

    def body(x_ref, w_ref, out_ref):
        xf = _unpack_tc(x_ref[:, :HW])
        y = jnp.dot(xf, w_ref[...], preferred_element_type=jnp.float32)
        out_ref[:, :HW] = _pack_tc(y)
        out_ref[:, HW:] = jnp.zeros((blk, HP - HW), jnp.int32)

    return pl.pallas_call(
        body,
        grid=(n // blk,),
        in_specs=[
            pl.BlockSpec((blk, HP), lambda i: (i, 0)),
            pl.BlockSpec((H, H), lambda i: (0, 0)),
        ],
        out_specs=pl.BlockSpec((blk, HP), lambda i: (i, 0)),
        out_shape=jax.ShapeDtypeStruct((n, HP), jnp.int32),
    )(x, W)


def _mm_out1(f_atoms, W_o1, b_o, blk=2000):
    na, fd = f_atoms.shape
    b2d = b_o.reshape(1, H)

    def body(fa_ref, w1_ref, b_ref, out_ref):
        out_ref[...] = jnp.dot(fa_ref[...], w1_ref[...],
                               preferred_element_type=jnp.float32) + b_ref[...]

    return pl.pallas_call(
        body,
        grid=(na // blk,),
        in_specs=[
            pl.BlockSpec((blk, fd), lambda i: (i, 0)),
            pl.BlockSpec((fd, H), lambda i: (0, 0)),
            pl.BlockSpec((1, H), lambda i: (0, 0)),
        ],
        out_specs=pl.BlockSpec((blk, H), lambda i: (i, 0)),
        out_shape=jax.ShapeDtypeStruct((na, H), jnp.float32),
    )(f_atoms, W_o1, b2d)


def _mm_out2(P, a_msg, W_o2, blk=2000):
    na = P.shape[0]

    def body(p_ref, am_ref, w2_ref, out_ref):
        am = _unpack_tc(am_ref[:, :HW])
        acc = p_ref[...] + jnp.dot(am, w2_ref[...],
                                   preferred_element_type=jnp.float32)
        out_ref[...] = jnp.maximum(acc, 0.0)

    return pl.pallas_call(
        body,
        grid=(na // blk,),
        in_specs=[
            pl.BlockSpec((blk, H), lambda i: (i, 0)),
            pl.BlockSpec((blk, HP), lambda i: (i, 0)),
            pl.BlockSpec((H, H), lambda i: (0, 0)),
        ],
        out_specs=pl.BlockSpec((blk, H), lambda i: (i, 0)),
        out_shape=jax.ShapeDtypeStruct((na, H), jnp.float32),
    )(P, a_msg, W_o2)


def _gather_sum(table, a2bT_flat, n_pad, max_nb, nc, ns, ac=80):
    nw = nc * ns
    cs = n_pad // (nw * ac)
    mesh = plsc.VectorSubcoreMesh(core_axis_name="c", subcore_axis_name="s")

    @functools.partial(
        pl.kernel,
        mesh=mesh,
        out_type=jax.ShapeDtypeStruct((n_pad, HP), jnp.int32),
        scratch_types=[
            pltpu.VMEM((ac,), jnp.int32),
            pltpu.VMEM((ac,), jnp.int32),
            pltpu.VMEM((ac, H), jnp.float32),
            pltpu.VMEM((ac, HP), jnp.int32),
            pltpu.VMEM((ac, HP), jnp.int32),
            pltpu.VMEM((ac, HP), jnp.int32),
            pltpu.SemaphoreType.DMA,
            pltpu.SemaphoreType.DMA,
            pltpu.SemaphoreType.DMA,
        ],
    )
    def k(table_hbm, a2bT_hbm, out_hbm, idx_0, idx_1, acc_v, rows_0,
          rows_1, out_v, sem_0, sem_1, sem_o):
        c = lax.axis_index("c")
        s = lax.axis_index("s")
        w = c * ns + s
        idx = (idx_0, idx_1)
        rows = (rows_0, rows_1)
        sems = (sem_0, sem_1)

        def chunk(kk, _):
            atom_base = (w * cs + kk) * ac

            for g in (0, 1):
                off = g * n_pad + atom_base
                pltpu.sync_copy(a2bT_hbm.at[pl.ds(off, ac)], idx[g])
                pltpu.async_copy(table_hbm.at[idx[g]], rows[g], sems[g])

            for g in range(max_nb):
                b = g % 2
                pltpu.make_async_copy(table_hbm.at[idx[b]], rows[b],
                                      sems[b]).wait()
                rbuf = rows[b]

                if g == 0:
                    def row0(r, _):
                        for d in range(HB):
                            v = rbuf[r, pl.ds(d * 16, 16)]
                            acc_v[r, pl.ds(d * 16, 16)] = _lo(v)
                            acc_v[r, pl.ds(HW + d * 16, 16)] = _hi(v)
                        return 0

                    lax.fori_loop(0, ac, row0, 0)
                else:
                    def rowa(r, _):
                        for d in range(HB):
                            v = rbuf[r, pl.ds(d * 16, 16)]
                            se = pl.ds(d * 16, 16)
                            so = pl.ds(HW + d * 16, 16)
                            acc_v[r, se] = acc_v[r, se] + _lo(v)
                            acc_v[r, so] = acc_v[r, so] + _hi(v)
                        return 0

                    lax.fori_loop(0, ac, rowa, 0)

                if g + 2 < max_nb:
                    off = (g + 2) * n_pad + atom_base
                    pltpu.sync_copy(a2bT_hbm.at[pl.ds(off, ac)], idx[b])
                    pltpu.async_copy(table_hbm.at[idx[b]], rows[b], sems[b])

            @pl.when(kk > 0)
            def _():
                pltpu.make_async_copy(out_v, out_hbm.at[pl.ds(0, ac)],
                                      sem_o).wait()

            def rowp(r, _):
                for d in range(HB):
                    e = acc_v[r, pl.ds(d * 16, 16)]
                    o = acc_v[r, pl.ds(HW + d * 16, 16)]
                    out_v[r, pl.ds(d * 16, 16)] = _pack(e, o)
                return 0

            lax.fori_loop(0, ac, rowp, 0)
            pltpu.async_copy(out_v, out_hbm.at[pl.ds(atom_base, ac)], sem_o)
            return 0

        lax.fori_loop(0, cs, chunk, 0)
        pltpu.make_async_copy(out_v, out_hbm.at[pl.ds(0, ac)], sem_o).wait()

    return k(table, a2bT_flat)


def _gather_msg(inp, mh, amh, b2a, b2revb, nc, ns, cr=40):
    nb = inp.shape[0]
    nw = nc * ns
    pw = nb // nw
    npair = pw // (2 * cr)
    mesh = plsc.VectorSubcoreMesh(core_axis_name="c", subcore_axis_name="s")

    @functools.partial(
        pl.kernel,
        mesh=mesh,
        out_type=jax.ShapeDtypeStruct((nb, HP), jnp.int32),
        scratch_types=[
            pltpu.VMEM((cr,), jnp.int32),
            pltpu.VMEM((cr,), jnp.int32),
            pltpu.VMEM((cr,), jnp.int32),
            pltpu.VMEM((cr,), jnp.int32),
            pltpu.VMEM((cr, HW), jnp.int32),
            pltpu.VMEM((cr, HW), jnp.int32),
            pltpu.VMEM((cr, HP), jnp.int32),
            pltpu.VMEM((cr, HP), jnp.int32),
            pltpu.VMEM((cr, HP), jnp.int32),
            pltpu.VMEM((cr, HP), jnp.int32),
            pltpu.VMEM((cr, HP), jnp.int32),
            pltpu.VMEM((cr, HP), jnp.int32),
            pltpu.SemaphoreType.DMA,
            pltpu.SemaphoreType.DMA,
            pltpu.SemaphoreType.DMA,
            pltpu.SemaphoreType.DMA,
        ],
    )
    def k(inp_hbm, mh_hbm, amh_hbm, b2a_hbm, b2revb_hbm, out_hbm,
          i1_0, i1_1, i2_0, i2_1, bi_0, bi_1, ba_0, ba_1, bb_0, bb_1,
          bo_0, bo_1, semi_0, semi_1, semo_0, semo_1):
        c = lax.axis_index("c")
        s = lax.axis_index("s")
        w = c * ns + s
        i1 = (i1_0, i1_1)
        i2 = (i2_0, i2_1)
        bi = (bi_0, bi_1)
        ba = (ba_0, ba_1)
        bb = (bb_0, bb_1)
        bo = (bo_0, bo_1)
        semi = (semi_0, semi_1)
        semo = (semo_0, semo_1)

        def issue_in(cc, sl):
            base = pl.multiple_of(w * pw + cc * cr, 8)
            pltpu.sync_copy(b2a_hbm.at[pl.ds(base, cr)], i1[sl])
            pltpu.sync_copy(b2revb_hbm.at[pl.ds(base, cr)], i2[sl])
            pltpu.async_copy(inp_hbm.at[pl.ds(base, cr)], bi[sl], semi[sl])
            pltpu.async_copy(amh_hbm.at[i1[sl]], ba[sl], semi[sl])
            pltpu.async_copy(mh_hbm.at[i2[sl]], bb[sl], semi[sl])

        def wait_in(sl):
            pltpu.make_async_copy(inp_hbm.at[pl.ds(0, cr)], bi[sl],
                                  semi[sl]).wait()
            pltpu.make_async_copy(amh_hbm.at[pl.ds(0, cr)], ba[sl],
                                  semi[sl]).wait()
            pltpu.make_async_copy(mh_hbm.at[pl.ds(0, cr)], bb[sl],
                                  semi[sl]).wait()

        def vpu(sl):
            bis, bas, bbs, bos = bi[sl], ba[sl], bb[sl], bo[sl]

            def row(r, _):
                for d in range(HB):
                    vi = bis[r, pl.ds(d * 16, 16)]
                    va = bas[r, pl.ds(d * 16, 16)]
                    vb = bbs[r, pl.ds(d * 16, 16)]
                    ve = jnp.maximum(_lo(vi) + _lo(va) - _lo(vb), 0.0)
                    vo = jnp.maximum(_hi(vi) + _hi(va) - _hi(vb), 0.0)
                    bos[r, pl.ds(d * 16, 16)] = _pack(ve, vo)
                return 0

            lax.fori_loop(0, cr, row, 0)

        def issue_out(cc, sl):
            base = pl.multiple_of(w * pw + cc * cr, 8)
            pltpu.async_copy(bo[sl], out_hbm.at[pl.ds(base, cr)], semo[sl])

        def wait_out(sl):
            pltpu.make_async_copy(bo[sl], out_hbm.at[pl.ds(0, cr)],
                                  semo[sl]).wait()

        issue_in(0, 0)

        def body(kk, _):
            c0 = 2 * kk
            c1 = 2 * kk + 1

            issue_in(c1, 1)
            wait_in(0)

            @pl.when(kk > 0)
            def _():
                wait_out(0)

            vpu(0)
            issue_out(c0, 0)
            wait_in(1)

            @pl.when(kk + 1 < npair)
            def _():
                issue_in(c0 + 2, 0)

            @pl.when(kk > 0)
            def _():
                wait_out(1)

            vpu(1)
            issue_out(c1, 1)
            return 0

        lax.fori_loop(0, npair, body, 0)
        wait_out(0)
        wait_out(1)

    return k(inp, mh, amh, b2a, b2revb)


def kernel(f_atoms, f_bonds, a2b, b2a, b2revb, W_i, W_h, W_o, b_o):
    n_atoms, max_nb = a2b.shape
    fd = f_atoms.shape[1]

    info = plsc.get_sparse_core_info()
    nc, ns = info.num_cores, info.num_subcores
    nw = nc * ns
    ac = 80

    grp = nw * ac
    n_pad = ((n_atoms + grp - 1) // grp) * grp

    b2a = b2a.astype(jnp.int32)
    b2revb = b2revb.astype(jnp.int32)
    a2bT_flat = jnp.pad(a2b.astype(jnp.int32),
                        ((0, n_pad - n_atoms), (0, 0))).T.reshape(-1)

    P = _mm_out1(f_atoms, W_o[:fd], b_o)
    inp_i, mh_i = _mm_first(f_bonds, W_i, W_h)
    amh_i = _gather_sum(mh_i, a2bT_flat, n_pad, max_nb, nc, ns, ac)
    msg_i = _gather_msg(inp_i, mh_i, amh_i, b2a, b2revb, nc, ns)
    for _ in range(DEPTH - 2):
        mh_i = _mm_matmul(msg_i, W_h)
        asum_i = _gather_sum(msg_i, a2bT_flat, n_pad, max_nb, nc, ns, ac)
        amh_i = _mm_matmul(asum_i, W_h)
        msg_i = _gather_msg(inp_i, mh_i, amh_i, b2a, b2revb, nc, ns)

    a_msg_i = _gather_sum(msg_i, a2bT_flat, n_pad, max_nb, nc, ns, ac)
    return _mm_out2(P, a_msg_i[:n_atoms], W_o[fd:])


# speedup vs baseline: 4.1994x; 1.1670x over previous
import functools

import jax
import jax.numpy as jnp
from jax import lax
from jax.experimental import pallas as pl
from jax.experimental.pallas import tpu as pltpu
from jax.experimental.pallas import tpu_sc as plsc

DEPTH = 4
H = 384
HW = H // 2
HP = 256
HB = H // 32


def _lo(v):
    return lax.bitcast_convert_type(v << 16, jnp.float32)


def _hi(v):
    return lax.bitcast_convert_type(v & jnp.int32(-65536), jnp.float32)


def _pack(e, o):
    ei = lax.bitcast_convert_type(e, jnp.int32)
    oi = lax.bitcast_convert_type(o, jnp.int32)
    ei = ei + jnp.int32(0x7FFF) + ((ei >> 16) & jnp.int32(1))
    oi = oi + jnp.int32(0x7FFF) + ((oi >> 16) & jnp.int32(1))
    return lax.shift_right_logical(ei, 16) | (oi & jnp.int32(-65536))


def _unpack_tc(v):
    lo = lax.bitcast_convert_type(v << 16, jnp.float32)
    hi = lax.bitcast_convert_type(v & jnp.int32(-65536), jnp.float32)
    return jnp.concatenate([lo, hi], axis=1)


def _pack_tc(y):
    return _pack(y[:, :HW], y[:, HW:])


def _mm_first(f_bonds, W_i, W_h, blk=1280):
    nb, bd = f_bonds.shape

    def body(fb_ref, wi_ref, wh_ref, inp_ref, mh_ref):
        inp = jnp.dot(fb_ref[...], wi_ref[...], preferred_element_type=jnp.float32)
        inp_ref[...] = _pack_tc(inp)
        mh = jnp.dot(jnp.maximum(inp, 0.0), wh_ref[...],
                     preferred_element_type=jnp.float32)
        mh_ref[:, :HW] = _pack_tc(mh)
        mh_ref[:, HW:] = jnp.zeros((blk, HP - HW), jnp.int32)

    return pl.pallas_call(
        body,
        grid=(nb // blk,),
        in_specs=[
            pl.BlockSpec((blk, bd), lambda i: (i, 0)),
            pl.BlockSpec((bd, H), lambda i: (0, 0)),
            pl.BlockSpec((H, H), lambda i: (0, 0)),
        ],
        out_specs=[
            pl.BlockSpec((blk, HW), lambda i: (i, 0)),
            pl.BlockSpec((blk, HP), lambda i: (i, 0)),
        ],
        out_shape=[
            jax.ShapeDtypeStruct((nb, HW), jnp.int32),
            jax.ShapeDtypeStruct((nb, HP), jnp.int32),
        ],
    )(f_bonds, W_i, W_h)


def _mm_matmul(x, W, blk=1280):
    n = x.shape[0]

    def body(x_ref, w_ref, out_ref):
        xf = _unpack_tc(x_ref[:, :HW])
        y = jnp.dot(xf, w_ref[...], preferred_element_type=jnp.float32)
        out_ref[:, :HW] = _pack_tc(y)
        out_ref[:, HW:] = jnp.zeros((blk, HP - HW), jnp.int32)

    return pl.pallas_call(
        body,
        grid=(n // blk,),
        in_specs=[
            pl.BlockSpec((blk, HP), lambda i: (i, 0)),
            pl.BlockSpec((H, H), lambda i: (0, 0)),
        ],
        out_specs=pl.BlockSpec((blk, HP), lambda i: (i, 0)),
        out_shape=jax.ShapeDtypeStruct((n, HP), jnp.int32),
    )(x, W)


def _mm_out1(f_atoms, W_o1, b_o, blk=2000):
    na, fd = f_atoms.shape
    b2d = b_o.reshape(1, H)

    def body(fa_ref, w1_ref, b_ref, out_ref):
        out_ref[...] = jnp.dot(fa_ref[...], w1_ref[...],
                               preferred_element_type=jnp.float32) + b_ref[...]

    return pl.pallas_call(
        body,
        grid=(na // blk,),
        in_specs=[
            pl.BlockSpec((blk, fd), lambda i: (i, 0)),
            pl.BlockSpec((fd, H), lambda i: (0, 0)),
            pl.BlockSpec((1, H), lambda i: (0, 0)),
        ],
        out_specs=pl.BlockSpec((blk, H), lambda i: (i, 0)),
        out_shape=jax.ShapeDtypeStruct((na, H), jnp.float32),
    )(f_atoms, W_o1, b2d)


def _mm_out2(P, a_msg, W_o2, blk=2000):
    na = P.shape[0]

    def body(p_ref, am_ref, w2_ref, out_ref):
        am = _unpack_tc(am_ref[:, :HW])
        acc = p_ref[...] + jnp.dot(am, w2_ref[...],
                                   preferred_element_type=jnp.float32)
        out_ref[...] = jnp.maximum(acc, 0.0)

    return pl.pallas_call(
        body,
        grid=(na // blk,),
        in_specs=[
            pl.BlockSpec((blk, H), lambda i: (i, 0)),
            pl.BlockSpec((blk, HP), lambda i: (i, 0)),
            pl.BlockSpec((H, H), lambda i: (0, 0)),
        ],
        out_specs=pl.BlockSpec((blk, H), lambda i: (i, 0)),
        out_shape=jax.ShapeDtypeStruct((na, H), jnp.float32),
    )(P, a_msg, W_o2)


def _gather_sum(table, a2b_flat, n_pad, max_nb, nc, ns, apc=4):
    nw = nc * ns
    rpc = apc * max_nb
    cs = n_pad // (nw * apc)
    npair = cs // 2
    mesh = plsc.VectorSubcoreMesh(core_axis_name="c", subcore_axis_name="s")

    @functools.partial(
        pl.kernel,
        mesh=mesh,
        out_type=jax.ShapeDtypeStruct((n_pad, HP), jnp.int32),
        scratch_types=[
            pltpu.VMEM((rpc,), jnp.int32),
            pltpu.VMEM((rpc,), jnp.int32),
            pltpu.VMEM((rpc, HP), jnp.int32),
            pltpu.VMEM((rpc, HP), jnp.int32),
            pltpu.VMEM((apc, HP), jnp.int32),
            pltpu.VMEM((apc, HP), jnp.int32),
            pltpu.SemaphoreType.DMA,
            pltpu.SemaphoreType.DMA,
            pltpu.SemaphoreType.DMA,
            pltpu.SemaphoreType.DMA,
        ],
    )
    def k(table_hbm, a2b_hbm, out_hbm, idx_0, idx_1, rows_0, rows_1,
          ov_0, ov_1, semi_0, semi_1, semo_0, semo_1):
        c = lax.axis_index("c")
        s = lax.axis_index("s")
        w = c * ns + s
        idx = (idx_0, idx_1)
        rows = (rows_0, rows_1)
        ov = (ov_0, ov_1)
        semi = (semi_0, semi_1)
        semo = (semo_0, semo_1)

        def issue_in(cc, sl):
            base = pl.multiple_of(((w * cs + cc) * apc) * max_nb, 8)
            pltpu.sync_copy(a2b_hbm.at[pl.ds(base, rpc)], idx[sl])
            pltpu.async_copy(table_hbm.at[idx[sl]], rows[sl], semi[sl])

        def wait_in(sl):
            pltpu.make_async_copy(table_hbm.at[idx[sl]], rows[sl],
                                  semi[sl]).wait()

        def vpu(sl):
            rbuf, obuf = rows[sl], ov[sl]

            def atom(a, _):
                base = a * max_nb
                for d in range(HB):
                    sl_ = pl.ds(d * 16, 16)
                    v = rbuf[base, sl_]
                    e = _lo(v)
                    o = _hi(v)
                    for j in range(1, max_nb):
                        vj = rbuf[base + j, sl_]
                        e = e + _lo(vj)
                        o = o + _hi(vj)
                    obuf[a, sl_] = _pack(e, o)
                return 0

            lax.fori_loop(0, apc, atom, 0)

        def issue_out(cc, sl):
            atom_base = (w * cs + cc) * apc
            pltpu.async_copy(ov[sl], out_hbm.at[pl.ds(atom_base, apc)],
                             semo[sl])

        def wait_out(sl):
            pltpu.make_async_copy(ov[sl], out_hbm.at[pl.ds(0, apc)],
                                  semo[sl]).wait()

        issue_in(0, 0)

        def body(kk, _):
            c0 = 2 * kk
            c1 = 2 * kk + 1

            issue_in(c1, 1)
            wait_in(0)

            @pl.when(kk > 0)
            def _():
                wait_out(0)

            vpu(0)
            issue_out(c0, 0)
            wait_in(1)

            @pl.when(kk + 1 < npair)
            def _():
                issue_in(c0 + 2, 0)

            @pl.when(kk > 0)
            def _():
                wait_out(1)

            vpu(1)
            issue_out(c1, 1)
            return 0

        lax.fori_loop(0, npair, body, 0)
        wait_out(0)
        wait_out(1)

    return k(table, a2b_flat)


def _gather_msg(inp, mh, amh, b2a, b2revb, nc, ns, cr=40):
    nb = inp.shape[0]
    nw = nc * ns
    pw = nb // nw
    npair = pw // (2 * cr)
    mesh = plsc.VectorSubcoreMesh(core_axis_name="c", subcore_axis_name="s")

    @functools.partial(
        pl.kernel,
        mesh=mesh,
        out_type=jax.ShapeDtypeStruct((nb, HP), jnp.int32),
        scratch_types=[
            pltpu.VMEM((cr,), jnp.int32),
            pltpu.VMEM((cr,), jnp.int32),
            pltpu.VMEM((cr,), jnp.int32),
            pltpu.VMEM((cr,), jnp.int32),
            pltpu.VMEM((cr, HW), jnp.int32),
            pltpu.VMEM((cr, HW), jnp.int32),
            pltpu.VMEM((cr, HP), jnp.int32),
            pltpu.VMEM((cr, HP), jnp.int32),
            pltpu.VMEM((cr, HP), jnp.int32),
            pltpu.VMEM((cr, HP), jnp.int32),
            pltpu.VMEM((cr, HP), jnp.int32),
            pltpu.VMEM((cr, HP), jnp.int32),
            pltpu.SemaphoreType.DMA,
            pltpu.SemaphoreType.DMA,
            pltpu.SemaphoreType.DMA,
            pltpu.SemaphoreType.DMA,
        ],
    )
    def k(inp_hbm, mh_hbm, amh_hbm, b2a_hbm, b2revb_hbm, out_hbm,
          i1_0, i1_1, i2_0, i2_1, bi_0, bi_1, ba_0, ba_1, bb_0, bb_1,
          bo_0, bo_1, semi_0, semi_1, semo_0, semo_1):
        c = lax.axis_index("c")
        s = lax.axis_index("s")
        w = c * ns + s
        i1 = (i1_0, i1_1)
        i2 = (i2_0, i2_1)
        bi = (bi_0, bi_1)
        ba = (ba_0, ba_1)
        bb = (bb_0, bb_1)
        bo = (bo_0, bo_1)
        semi = (semi_0, semi_1)
        semo = (semo_0, semo_1)

        def issue_in(cc, sl):
            base = pl.multiple_of(w * pw + cc * cr, 8)
            pltpu.sync_copy(b2a_hbm.at[pl.ds(base, cr)], i1[sl])
            pltpu.sync_copy(b2revb_hbm.at[pl.ds(base, cr)], i2[sl])
            pltpu.async_copy(inp_hbm.at[pl.ds(base, cr)], bi[sl], semi[sl])
            pltpu.async_copy(amh_hbm.at[i1[sl]], ba[sl], semi[sl])
            pltpu.async_copy(mh_hbm.at[i2[sl]], bb[sl], semi[sl])

        def wait_in(sl):
            pltpu.make_async_copy(inp_hbm.at[pl.ds(0, cr)], bi[sl],
                                  semi[sl]).wait()
            pltpu.make_async_copy(amh_hbm.at[pl.ds(0, cr)], ba[sl],
                                  semi[sl]).wait()
            pltpu.make_async_copy(mh_hbm.at[pl.ds(0, cr)], bb[sl],
                                  semi[sl]).wait()

        def vpu(sl):
            bis, bas, bbs, bos = bi[sl], ba[sl], bb[sl], bo[sl]

            def row(r, _):
                for d in range(HB):
                    vi = bis[r, pl.ds(d * 16, 16)]
                    va = bas[r, pl.ds(d * 16, 16)]
                    vb = bbs[r, pl.ds(d * 16, 16)]
                    ve = jnp.maximum(_lo(vi) + _lo(va) - _lo(vb), 0.0)
                    vo = jnp.maximum(_hi(vi) + _hi(va) - _hi(vb), 0.0)
                    bos[r, pl.ds(d * 16, 16)] = _pack(ve, vo)
                return 0

            lax.fori_loop(0, cr, row, 0)

        def issue_out(cc, sl):
            base = pl.multiple_of(w * pw + cc * cr, 8)
            pltpu.async_copy(bo[sl], out_hbm.at[pl.ds(base, cr)], semo[sl])

        def wait_out(sl):
            pltpu.make_async_copy(bo[sl], out_hbm.at[pl.ds(0, cr)],
                                  semo[sl]).wait()

        issue_in(0, 0)

        def body(kk, _):
            c0 = 2 * kk
            c1 = 2 * kk + 1

            issue_in(c1, 1)
            wait_in(0)

            @pl.when(kk > 0)
            def _():
                wait_out(0)

            vpu(0)
            issue_out(c0, 0)
            wait_in(1)

            @pl.when(kk + 1 < npair)
            def _():
                issue_in(c0 + 2, 0)

            @pl.when(kk > 0)
            def _():
                wait_out(1)

            vpu(1)
            issue_out(c1, 1)
            return 0

        lax.fori_loop(0, npair, body, 0)
        wait_out(0)
        wait_out(1)

    return k(inp, mh, amh, b2a, b2revb)


def kernel(f_atoms, f_bonds, a2b, b2a, b2revb, W_i, W_h, W_o, b_o):
    n_atoms, max_nb = a2b.shape
    fd = f_atoms.shape[1]

    info = plsc.get_sparse_core_info()
    nc, ns = info.num_cores, info.num_subcores
    nw = nc * ns
    apc = 4

    grp = nw * apc * 2
    n_pad = ((n_atoms + grp - 1) // grp) * grp

    b2a = b2a.astype(jnp.int32)
    b2revb = b2revb.astype(jnp.int32)
    a2b_flat = jnp.pad(a2b.astype(jnp.int32),
                       ((0, n_pad - n_atoms), (0, 0))).reshape(-1)

    P = _mm_out1(f_atoms, W_o[:fd], b_o)
    inp_i, mh_i = _mm_first(f_bonds, W_i, W_h)
    amh_i = _gather_sum(mh_i, a2b_flat, n_pad, max_nb, nc, ns, apc)
    msg_i = _gather_msg(inp_i, mh_i, amh_i, b2a, b2revb, nc, ns)
    for _ in range(DEPTH - 2):
        mh_i = _mm_matmul(msg_i, W_h)
        asum_i = _gather_sum(msg_i, a2b_flat, n_pad, max_nb, nc, ns, apc)
        amh_i = _mm_matmul(asum_i, W_h)
        msg_i = _gather_msg(inp_i, mh_i, amh_i, b2a, b2revb, nc, ns)

    a_msg_i = _gather_sum(msg_i, a2b_flat, n_pad, max_nb, nc, ns, apc)
    return _mm_out2(P, a_msg_i[:n_atoms], W_o[fd:])


# speedup vs baseline: 4.8482x; 1.1545x over previous
import functools

import jax
import jax.numpy as jnp
from jax import lax
from jax.experimental import pallas as pl
from jax.experimental.pallas import tpu as pltpu
from jax.experimental.pallas import tpu_sc as plsc

DEPTH = 4
H = 384
HW = H // 2
HP = 256
HB = H // 32


def _lo(v):
    return lax.bitcast_convert_type(v << 16, jnp.float32)


def _hi(v):
    return lax.bitcast_convert_type(v & jnp.int32(-65536), jnp.float32)


def _pack(e, o):
    ei = lax.bitcast_convert_type(e, jnp.int32)
    oi = lax.bitcast_convert_type(o, jnp.int32)
    ei = ei + jnp.int32(0x7FFF) + ((ei >> 16) & jnp.int32(1))
    oi = oi + jnp.int32(0x7FFF) + ((oi >> 16) & jnp.int32(1))
    return lax.shift_right_logical(ei, 16) | (oi & jnp.int32(-65536))


def _unpack_tc(v):
    lo = lax.bitcast_convert_type(v << 16, jnp.float32)
    hi = lax.bitcast_convert_type(v & jnp.int32(-65536), jnp.float32)
    return jnp.concatenate([lo, hi], axis=1)


def _pack_tc(y):
    return _pack(y[:, :HW], y[:, HW:])


def _mm_first(f_bonds, W_i, W_h, blk=1280):
    nb, bd = f_bonds.shape

    def body(fb_ref, wi_ref, wh_ref, inp_ref, mh_ref):
        inp = jnp.dot(fb_ref[...], wi_ref[...], preferred_element_type=jnp.float32)
        inp_ref[...] = _pack_tc(inp)
        mh = jnp.dot(jnp.maximum(inp, 0.0), wh_ref[...],
                     preferred_element_type=jnp.float32)
        mh_ref[:, :HW] = _pack_tc(mh)
        mh_ref[:, HW:] = jnp.zeros((blk, HP - HW), jnp.int32)

    return pl.pallas_call(
        body,
        grid=(nb // blk,),
        in_specs=[
            pl.BlockSpec((blk, bd), lambda i: (i, 0)),
            pl.BlockSpec((bd, H), lambda i: (0, 0)),
            pl.BlockSpec((H, H), lambda i: (0, 0)),
        ],
        out_specs=[
            pl.BlockSpec((blk, HW), lambda i: (i, 0)),
            pl.BlockSpec((blk, HP), lambda i: (i, 0)),
        ],
        out_shape=[
            jax.ShapeDtypeStruct((nb, HW), jnp.int32),
            jax.ShapeDtypeStruct((nb, HP), jnp.int32),
        ],
    )(f_bonds, W_i, W_h)


def _mm_matmul(x, W, blk=1280):
    n = x.shape[0]

    def body(x_ref, w_ref, out_ref):
        xf = _unpack_tc(x_ref[:, :HW])
        y = jnp.dot(xf, w_ref[...], preferred_element_type=jnp.float32)
        out_ref[:, :HW] = _pack_tc(y)
        out_ref[:, HW:] = jnp.zeros((blk, HP - HW), jnp.int32)

    return pl.pallas_call(
        body,
        grid=(n // blk,),
        in_specs=[
            pl.BlockSpec((blk, HP), lambda i: (i, 0)),
            pl.BlockSpec((H, H), lambda i: (0, 0)),
        ],
        out_specs=pl.BlockSpec((blk, HP), lambda i: (i, 0)),
        out_shape=jax.ShapeDtypeStruct((n, HP), jnp.int32),
    )(x, W)


def _mm_out1(f_atoms, W_o1, b_o, blk=2000):
    na, fd = f_atoms.shape
    b2d = b_o.reshape(1, H)

    def body(fa_ref, w1_ref, b_ref, out_ref):
        out_ref[...] = jnp.dot(fa_ref[...], w1_ref[...],
                               preferred_element_type=jnp.float32) + b_ref[...]

    return pl.pallas_call(
        body,
        grid=(na // blk,),
        in_specs=[
            pl.BlockSpec((blk, fd), lambda i: (i, 0)),
            pl.BlockSpec((fd, H), lambda i: (0, 0)),
            pl.BlockSpec((1, H), lambda i: (0, 0)),
        ],
        out_specs=pl.BlockSpec((blk, H), lambda i: (i, 0)),
        out_shape=jax.ShapeDtypeStruct((na, H), jnp.float32),
    )(f_atoms, W_o1, b2d)


def _mm_out2(P, a_msg, W_o2, blk=2000):
    na = P.shape[0]

    def body(p_ref, am_ref, w2_ref, out_ref):
        am = _unpack_tc(am_ref[:, :HW])
        acc = p_ref[...] + jnp.dot(am, w2_ref[...],
                                   preferred_element_type=jnp.float32)
        out_ref[...] = jnp.maximum(acc, 0.0)

    return pl.pallas_call(
        body,
        grid=(na // blk,),
        in_specs=[
            pl.BlockSpec((blk, H), lambda i: (i, 0)),
            pl.BlockSpec((blk, HP), lambda i: (i, 0)),
            pl.BlockSpec((H, H), lambda i: (0, 0)),
        ],
        out_specs=pl.BlockSpec((blk, H), lambda i: (i, 0)),
        out_shape=jax.ShapeDtypeStruct((na, H), jnp.float32),
    )(P, a_msg, W_o2)


def _gather_sum(table, a2b_flat, n_pad, max_nb, nc, ns, apc=4):
    nw = nc * ns
    rpc = apc * max_nb
    cs = n_pad // (nw * apc)
    npair = cs // 2
    mesh = plsc.VectorSubcoreMesh(core_axis_name="c", subcore_axis_name="s")

    @functools.partial(
        pl.kernel,
        mesh=mesh,
        out_type=jax.ShapeDtypeStruct((n_pad, HP), jnp.int32),
        scratch_types=[
            pltpu.VMEM((cs * rpc,), jnp.int32),
            pltpu.VMEM((rpc, HP), jnp.int32),
            pltpu.VMEM((rpc, HP), jnp.int32),
            pltpu.VMEM((apc, HP), jnp.int32),
            pltpu.VMEM((apc, HP), jnp.int32),
            pltpu.SemaphoreType.DMA,
            pltpu.SemaphoreType.DMA,
            pltpu.SemaphoreType.DMA,
            pltpu.SemaphoreType.DMA,
        ],
    )
    def k(table_hbm, a2b_hbm, out_hbm, idx_all, rows_0, rows_1,
          ov_0, ov_1, semi_0, semi_1, semo_0, semo_1):
        c = lax.axis_index("c")
        s = lax.axis_index("s")
        w = c * ns + s
        rows = (rows_0, rows_1)
        ov = (ov_0, ov_1)
        semi = (semi_0, semi_1)
        semo = (semo_0, semo_1)

        pltpu.sync_copy(a2b_hbm.at[pl.ds(w * cs * rpc, cs * rpc)], idx_all)

        def issue_in(cc, sl):
            off = pl.multiple_of(cc * rpc, 8)
            pltpu.async_copy(table_hbm.at[idx_all.at[pl.ds(off, rpc)]],
                             rows[sl], semi[sl])

        def wait_in(sl):
            pltpu.make_async_copy(table_hbm.at[idx_all.at[pl.ds(0, rpc)]],
                                  rows[sl], semi[sl]).wait()

        def vpu(sl):
            rbuf, obuf = rows[sl], ov[sl]

            def atom(a, _):
                base = a * max_nb
                for d in range(HB):
                    sl_ = pl.ds(d * 16, 16)
                    v = rbuf[base, sl_]
                    e = _lo(v)
                    o = _hi(v)
                    for j in range(1, max_nb):
                        vj = rbuf[base + j, sl_]
                        e = e + _lo(vj)
                        o = o + _hi(vj)
                    obuf[a, sl_] = _pack(e, o)
                return 0

            lax.fori_loop(0, apc, atom, 0)

        def issue_out(cc, sl):
            atom_base = (w * cs + cc) * apc
            pltpu.async_copy(ov[sl], out_hbm.at[pl.ds(atom_base, apc)],
                             semo[sl])

        def wait_out(sl):
            pltpu.make_async_copy(ov[sl], out_hbm.at[pl.ds(0, apc)],
                                  semo[sl]).wait()

        issue_in(0, 0)

        def body(kk, _):
            c0 = 2 * kk
            c1 = 2 * kk + 1

            issue_in(c1, 1)
            wait_in(0)

            @pl.when(kk > 0)
            def _():
                wait_out(0)

            vpu(0)
            issue_out(c0, 0)
            wait_in(1)

            @pl.when(kk + 1 < npair)
            def _():
                issue_in(c0 + 2, 0)

            @pl.when(kk > 0)
            def _():
                wait_out(1)

            vpu(1)
            issue_out(c1, 1)
            return 0

        lax.fori_loop(0, npair, body, 0)
        wait_out(0)
        wait_out(1)

    return k(table, a2b_flat)


def _gather_msg(inp, mh, amh, b2a, b2revb, nc, ns, cr=40):
    nb = inp.shape[0]
    nw = nc * ns
    pw = nb // nw
    npair = pw // (2 * cr)
    mesh = plsc.VectorSubcoreMesh(core_axis_name="c", subcore_axis_name="s")

    @functools.partial(
        pl.kernel,
        mesh=mesh,
        out_type=jax.ShapeDtypeStruct((nb, HP), jnp.int32),
        scratch_types=[
            pltpu.VMEM((pw,), jnp.int32),
            pltpu.VMEM((pw,), jnp.int32),
            pltpu.VMEM((cr, HW), jnp.int32),
            pltpu.VMEM((cr, HW), jnp.int32),
            pltpu.VMEM((cr, HP), jnp.int32),
            pltpu.VMEM((cr, HP), jnp.int32),
            pltpu.VMEM((cr, HP), jnp.int32),
            pltpu.VMEM((cr, HP), jnp.int32),
            pltpu.VMEM((cr, HP), jnp.int32),
            pltpu.VMEM((cr, HP), jnp.int32),
            pltpu.SemaphoreType.DMA,
            pltpu.SemaphoreType.DMA,
            pltpu.SemaphoreType.DMA,
            pltpu.SemaphoreType.DMA,
        ],
    )
    def k(inp_hbm, mh_hbm, amh_hbm, b2a_hbm, b2revb_hbm, out_hbm,
          i1_all, i2_all, bi_0, bi_1, ba_0, ba_1, bb_0, bb_1,
          bo_0, bo_1, semi_0, semi_1, semo_0, semo_1):
        c = lax.axis_index("c")
        s = lax.axis_index("s")
        w = c * ns + s
        bi = (bi_0, bi_1)
        ba = (ba_0, ba_1)
        bb = (bb_0, bb_1)
        bo = (bo_0, bo_1)
        semi = (semi_0, semi_1)
        semo = (semo_0, semo_1)

        pltpu.sync_copy(b2a_hbm.at[pl.ds(w * pw, pw)], i1_all)
        pltpu.sync_copy(b2revb_hbm.at[pl.ds(w * pw, pw)], i2_all)

        def issue_in(cc, sl):
            base = pl.multiple_of(w * pw + cc * cr, 8)
            off = pl.multiple_of(cc * cr, 8)
            pltpu.async_copy(inp_hbm.at[pl.ds(base, cr)], bi[sl], semi[sl])
            pltpu.async_copy(amh_hbm.at[i1_all.at[pl.ds(off, cr)]],
                             ba[sl], semi[sl])
            pltpu.async_copy(mh_hbm.at[i2_all.at[pl.ds(off, cr)]],
                             bb[sl], semi[sl])

        def wait_in(sl):
            pltpu.make_async_copy(inp_hbm.at[pl.ds(0, cr)], bi[sl],
                                  semi[sl]).wait()
            pltpu.make_async_copy(amh_hbm.at[i1_all.at[pl.ds(0, cr)]],
                                  ba[sl], semi[sl]).wait()
            pltpu.make_async_copy(mh_hbm.at[i2_all.at[pl.ds(0, cr)]],
                                  bb[sl], semi[sl]).wait()

        def vpu(sl):
            bis, bas, bbs, bos = bi[sl], ba[sl], bb[sl], bo[sl]

            def row(r, _):
                for d in range(HB):
                    vi = bis[r, pl.ds(d * 16, 16)]
                    va = bas[r, pl.ds(d * 16, 16)]
                    vb = bbs[r, pl.ds(d * 16, 16)]
                    ve = jnp.maximum(_lo(vi) + _lo(va) - _lo(vb), 0.0)
                    vo = jnp.maximum(_hi(vi) + _hi(va) - _hi(vb), 0.0)
                    bos[r, pl.ds(d * 16, 16)] = _pack(ve, vo)
                return 0

            lax.fori_loop(0, cr, row, 0)

        def issue_out(cc, sl):
            base = pl.multiple_of(w * pw + cc * cr, 8)
            pltpu.async_copy(bo[sl], out_hbm.at[pl.ds(base, cr)], semo[sl])

        def wait_out(sl):
            pltpu.make_async_copy(bo[sl], out_hbm.at[pl.ds(0, cr)],
                                  semo[sl]).wait()

        issue_in(0, 0)

        def body(kk, _):
            c0 = 2 * kk
            c1 = 2 * kk + 1

            issue_in(c1, 1)
            wait_in(0)

            @pl.when(kk > 0)
            def _():
                wait_out(0)

            vpu(0)
            issue_out(c0, 0)
            wait_in(1)

            @pl.when(kk + 1 < npair)
            def _():
                issue_in(c0 + 2, 0)

            @pl.when(kk > 0)
            def _():
                wait_out(1)

            vpu(1)
            issue_out(c1, 1)
            return 0

        lax.fori_loop(0, npair, body, 0)
        wait_out(0)
        wait_out(1)

    return k(inp, mh, amh, b2a, b2revb)


def kernel(f_atoms, f_bonds, a2b, b2a, b2revb, W_i, W_h, W_o, b_o):
    n_atoms, max_nb = a2b.shape
    fd = f_atoms.shape[1]

    info = plsc.get_sparse_core_info()
    nc, ns = info.num_cores, info.num_subcores
    nw = nc * ns
    apc = 4

    grp = nw * apc * 2
    n_pad = ((n_atoms + grp - 1) // grp) * grp

    b2a = b2a.astype(jnp.int32)
    b2revb = b2revb.astype(jnp.int32)
    a2b_flat = jnp.pad(a2b.astype(jnp.int32),
                       ((0, n_pad - n_atoms), (0, 0))).reshape(-1)

    P = _mm_out1(f_atoms, W_o[:fd], b_o)
    inp_i, mh_i = _mm_first(f_bonds, W_i, W_h)
    amh_i = _gather_sum(mh_i, a2b_flat, n_pad, max_nb, nc, ns, apc)
    msg_i = _gather_msg(inp_i, mh_i, amh_i, b2a, b2revb, nc, ns)
    for _ in range(DEPTH - 2):
        mh_i = _mm_matmul(msg_i, W_h)
        asum_i = _gather_sum(msg_i, a2b_flat, n_pad, max_nb, nc, ns, apc)
        amh_i = _mm_matmul(asum_i, W_h)
        msg_i = _gather_msg(inp_i, mh_i, amh_i, b2a, b2revb, nc, ns)

    a_msg_i = _gather_sum(msg_i, a2b_flat, n_pad, max_nb, nc, ns, apc)
    return _mm_out2(P, a_msg_i[:n_atoms], W_o[fd:])


# speedup vs baseline: 4.8719x; 1.0049x over previous
import functools

import jax
import jax.numpy as jnp
from jax import lax
from jax.experimental import pallas as pl
from jax.experimental.pallas import tpu as pltpu
from jax.experimental.pallas import tpu_sc as plsc

DEPTH = 4
H = 384
HW = H // 2
HP = 256
HB = H // 32


def _lo(v):
    return lax.bitcast_convert_type(v << 16, jnp.float32)


def _hi(v):
    return lax.bitcast_convert_type(v & jnp.int32(-65536), jnp.float32)


def _pack(e, o):
    ei = lax.bitcast_convert_type(e, jnp.int32)
    oi = lax.bitcast_convert_type(o, jnp.int32)
    ei = ei + jnp.int32(0x7FFF) + ((ei >> 16) & jnp.int32(1))
    oi = oi + jnp.int32(0x7FFF) + ((oi >> 16) & jnp.int32(1))
    return lax.shift_right_logical(ei, 16) | (oi & jnp.int32(-65536))


def _unpack_tc(v):
    lo = lax.bitcast_convert_type(v << 16, jnp.float32)
    hi = lax.bitcast_convert_type(v & jnp.int32(-65536), jnp.float32)
    return jnp.concatenate([lo, hi], axis=1)


def _pack_tc(y):
    return _pack(y[:, :HW], y[:, HW:])


def _mm_first(f_bonds, W_i, W_h, blk=1280):
    nb, bd = f_bonds.shape

    def body(fb_ref, wi_ref, wh_ref, inp_ref, mh_ref):
        inp = jnp.dot(fb_ref[...], wi_ref[...], preferred_element_type=jnp.float32)
        inp_ref[...] = _pack_tc(inp)
        mh = jnp.dot(jnp.maximum(inp, 0.0), wh_ref[...],
                     preferred_element_type=jnp.float32)
        mh_ref[:, :HW] = _pack_tc(mh)
        mh_ref[:, HW:] = jnp.zeros((blk, HP - HW), jnp.int32)

    return pl.pallas_call(
        body,
        grid=(nb // blk,),
        in_specs=[
            pl.BlockSpec((blk, bd), lambda i: (i, 0)),
            pl.BlockSpec((bd, H), lambda i: (0, 0)),
            pl.BlockSpec((H, H), lambda i: (0, 0)),
        ],
        out_specs=[
            pl.BlockSpec((blk, HW), lambda i: (i, 0)),
            pl.BlockSpec((blk, HP), lambda i: (i, 0)),
        ],
        out_shape=[
            jax.ShapeDtypeStruct((nb, HW), jnp.int32),
            jax.ShapeDtypeStruct((nb, HP), jnp.int32),
        ],
    )(f_bonds, W_i, W_h)


def _mm_matmul(x, W, blk=1280):
    n = x.shape[0]

    def body(x_ref, w_ref, out_ref):
        xf = _unpack_tc(x_ref[:, :HW])
        y = jnp.dot(xf, w_ref[...], preferred_element_type=jnp.float32)
        out_ref[:, :HW] = _pack_tc(y)
        out_ref[:, HW:] = jnp.zeros((blk, HP - HW), jnp.int32)

    return pl.pallas_call(
        body,
        grid=(n // blk,),
        in_specs=[
            pl.BlockSpec((blk, HP), lambda i: (i, 0)),
            pl.BlockSpec((H, H), lambda i: (0, 0)),
        ],
        out_specs=pl.BlockSpec((blk, HP), lambda i: (i, 0)),
        out_shape=jax.ShapeDtypeStruct((n, HP), jnp.int32),
    )(x, W)


def _mm_out1(f_atoms, W_o1, b_o, blk=2000):
    na, fd = f_atoms.shape
    b2d = b_o.reshape(1, H)

    def body(fa_ref, w1_ref, b_ref, out_ref):
        out_ref[...] = jnp.dot(fa_ref[...], w1_ref[...],
                               preferred_element_type=jnp.float32) + b_ref[...]

    return pl.pallas_call(
        body,
        grid=(na // blk,),
        in_specs=[
            pl.BlockSpec((blk, fd), lambda i: (i, 0)),
            pl.BlockSpec((fd, H), lambda i: (0, 0)),
            pl.BlockSpec((1, H), lambda i: (0, 0)),
        ],
        out_specs=pl.BlockSpec((blk, H), lambda i: (i, 0)),
        out_shape=jax.ShapeDtypeStruct((na, H), jnp.float32),
    )(f_atoms, W_o1, b2d)


def _mm_out2(P, a_msg, W_o2, blk=2000):
    na = P.shape[0]

    def body(p_ref, am_ref, w2_ref, out_ref):
        am = _unpack_tc(am_ref[:, :HW])
        acc = p_ref[...] + jnp.dot(am, w2_ref[...],
                                   preferred_element_type=jnp.float32)
        out_ref[...] = jnp.maximum(acc, 0.0)

    return pl.pallas_call(
        body,
        grid=(na // blk,),
        in_specs=[
            pl.BlockSpec((blk, H), lambda i: (i, 0)),
            pl.BlockSpec((blk, HP), lambda i: (i, 0)),
            pl.BlockSpec((H, H), lambda i: (0, 0)),
        ],
        out_specs=pl.BlockSpec((blk, H), lambda i: (i, 0)),
        out_shape=jax.ShapeDtypeStruct((na, H), jnp.float32),
    )(P, a_msg, W_o2)


def _gather_sum(table, a2b_flat, n_pad, max_nb, nc, ns, apc=4):
    nw = nc * ns
    rpc = apc * max_nb
    cs = n_pad // (nw * apc)
    nq = cs // 4
    mesh = plsc.VectorSubcoreMesh(core_axis_name="c", subcore_axis_name="s")

    @functools.partial(
        pl.kernel,
        mesh=mesh,
        out_type=jax.ShapeDtypeStruct((n_pad, HP), jnp.int32),
        scratch_types=[
            pltpu.VMEM((cs * rpc,), jnp.int32),
            pltpu.VMEM((rpc, HP), jnp.int32),
            pltpu.VMEM((rpc, HP), jnp.int32),
            pltpu.VMEM((rpc, HP), jnp.int32),
            pltpu.VMEM((rpc, HP), jnp.int32),
            pltpu.VMEM((apc, HP), jnp.int32),
            pltpu.VMEM((apc, HP), jnp.int32),
            pltpu.VMEM((apc, HP), jnp.int32),
            pltpu.VMEM((apc, HP), jnp.int32),
            pltpu.SemaphoreType.DMA,
            pltpu.SemaphoreType.DMA,
            pltpu.SemaphoreType.DMA,
            pltpu.SemaphoreType.DMA,
            pltpu.SemaphoreType.DMA,
            pltpu.SemaphoreType.DMA,
            pltpu.SemaphoreType.DMA,
            pltpu.SemaphoreType.DMA,
        ],
    )
    def k(table_hbm, a2b_hbm, out_hbm, idx_all, rows_0, rows_1, rows_2,
          rows_3, ov_0, ov_1, ov_2, ov_3, semi_0, semi_1, semi_2, semi_3,
          semo_0, semo_1, semo_2, semo_3):
        c = lax.axis_index("c")
        s = lax.axis_index("s")
        w = c * ns + s
        rows = (rows_0, rows_1, rows_2, rows_3)
        ov = (ov_0, ov_1, ov_2, ov_3)
        semi = (semi_0, semi_1, semi_2, semi_3)
        semo = (semo_0, semo_1, semo_2, semo_3)

        pltpu.sync_copy(a2b_hbm.at[pl.ds(w * cs * rpc, cs * rpc)], idx_all)

        def issue_in(cc, sl):
            off = pl.multiple_of(cc * rpc, 8)
            pltpu.async_copy(table_hbm.at[idx_all.at[pl.ds(off, rpc)]],
                             rows[sl], semi[sl])

        def wait_in(sl):
            pltpu.make_async_copy(table_hbm.at[idx_all.at[pl.ds(0, rpc)]],
                                  rows[sl], semi[sl]).wait()

        def vpu(sl):
            rbuf, obuf = rows[sl], ov[sl]

            def atom(a, _):
                base = a * max_nb
                for d in range(HB):
                    sl_ = pl.ds(d * 16, 16)
                    v = rbuf[base, sl_]
                    e = _lo(v)
                    o = _hi(v)
                    for j in range(1, max_nb):
                        vj = rbuf[base + j, sl_]
                        e = e + _lo(vj)
                        o = o + _hi(vj)
                    obuf[a, sl_] = _pack(e, o)
                return 0

            lax.fori_loop(0, apc, atom, 0)

        def issue_out(cc, sl):
            atom_base = (w * cs + cc) * apc
            pltpu.async_copy(ov[sl], out_hbm.at[pl.ds(atom_base, apc)],
                             semo[sl])

        def wait_out(sl):
            pltpu.make_async_copy(ov[sl], out_hbm.at[pl.ds(0, apc)],
                                  semo[sl]).wait()

        for sl in (0, 1, 2):
            issue_in(sl, sl)

        def body(kk, _):
            for b in range(4):
                cc = 4 * kk + b
                nxt = cc + 3

                @pl.when(nxt < cs)
                def _():
                    issue_in(nxt, (b + 3) % 4)

                wait_in(b)

                @pl.when(kk > 0)
                def _():
                    wait_out(b)

                vpu(b)
                issue_out(cc, b)
            return 0

        lax.fori_loop(0, nq, body, 0)
        for sl in (0, 1, 2, 3):
            wait_out(sl)

    return k(table, a2b_flat)


def _gather_msg(inp, mh, amh, b2a, b2revb, nc, ns, cr=40):
    nb = inp.shape[0]
    nw = nc * ns
    pw = nb // nw
    npair = pw // (2 * cr)
    mesh = plsc.VectorSubcoreMesh(core_axis_name="c", subcore_axis_name="s")

    @functools.partial(
        pl.kernel,
        mesh=mesh,
        out_type=jax.ShapeDtypeStruct((nb, HP), jnp.int32),
        scratch_types=[
            pltpu.VMEM((pw,), jnp.int32),
            pltpu.VMEM((pw,), jnp.int32),
            pltpu.VMEM((cr, HW), jnp.int32),
            pltpu.VMEM((cr, HW), jnp.int32),
            pltpu.VMEM((cr, HP), jnp.int32),
            pltpu.VMEM((cr, HP), jnp.int32),
            pltpu.VMEM((cr, HP), jnp.int32),
            pltpu.VMEM((cr, HP), jnp.int32),
            pltpu.VMEM((cr, HP), jnp.int32),
            pltpu.VMEM((cr, HP), jnp.int32),
            pltpu.SemaphoreType.DMA,
            pltpu.SemaphoreType.DMA,
            pltpu.SemaphoreType.DMA,
            pltpu.SemaphoreType.DMA,
        ],
    )
    def k(inp_hbm, mh_hbm, amh_hbm, b2a_hbm, b2revb_hbm, out_hbm,
          i1_all, i2_all, bi_0, bi_1, ba_0, ba_1, bb_0, bb_1,
          bo_0, bo_1, semi_0, semi_1, semo_0, semo_1):
        c = lax.axis_index("c")
        s = lax.axis_index("s")
        w = c * ns + s
        bi = (bi_0, bi_1)
        ba = (ba_0, ba_1)
        bb = (bb_0, bb_1)
        bo = (bo_0, bo_1)
        semi = (semi_0, semi_1)
        semo = (semo_0, semo_1)

        pltpu.sync_copy(b2a_hbm.at[pl.ds(w * pw, pw)], i1_all)
        pltpu.sync_copy(b2revb_hbm.at[pl.ds(w * pw, pw)], i2_all)

        def issue_in(cc, sl):
            base = pl.multiple_of(w * pw + cc * cr, 8)
            off = pl.multiple_of(cc * cr, 8)
            pltpu.async_copy(inp_hbm.at[pl.ds(base, cr)], bi[sl], semi[sl])
            pltpu.async_copy(amh_hbm.at[i1_all.at[pl.ds(off, cr)]],
                             ba[sl], semi[sl])
            pltpu.async_copy(mh_hbm.at[i2_all.at[pl.ds(off, cr)]],
                             bb[sl], semi[sl])

        def wait_in(sl):
            pltpu.make_async_copy(inp_hbm.at[pl.ds(0, cr)], bi[sl],
                                  semi[sl]).wait()
            pltpu.make_async_copy(amh_hbm.at[i1_all.at[pl.ds(0, cr)]],
                                  ba[sl], semi[sl]).wait()
            pltpu.make_async_copy(mh_hbm.at[i2_all.at[pl.ds(0, cr)]],
                                  bb[sl], semi[sl]).wait()

        def vpu(sl):
            bis, bas, bbs, bos = bi[sl], ba[sl], bb[sl], bo[sl]

            def row(r, _):
                for d in range(HB):
                    vi = bis[r, pl.ds(d * 16, 16)]
                    va = bas[r, pl.ds(d * 16, 16)]
                    vb = bbs[r, pl.ds(d * 16, 16)]
                    ve = jnp.maximum(_lo(vi) + _lo(va) - _lo(vb), 0.0)
                    vo = jnp.maximum(_hi(vi) + _hi(va) - _hi(vb), 0.0)
                    bos[r, pl.ds(d * 16, 16)] = _pack(ve, vo)
                return 0

            lax.fori_loop(0, cr, row, 0)

        def issue_out(cc, sl):
            base = pl.multiple_of(w * pw + cc * cr, 8)
            pltpu.async_copy(bo[sl], out_hbm.at[pl.ds(base, cr)], semo[sl])

        def wait_out(sl):
            pltpu.make_async_copy(bo[sl], out_hbm.at[pl.ds(0, cr)],
                                  semo[sl]).wait()

        issue_in(0, 0)

        def body(kk, _):
            c0 = 2 * kk
            c1 = 2 * kk + 1

            issue_in(c1, 1)
            wait_in(0)

            @pl.when(kk > 0)
            def _():
                wait_out(0)

            vpu(0)
            issue_out(c0, 0)
            wait_in(1)

            @pl.when(kk + 1 < npair)
            def _():
                issue_in(c0 + 2, 0)

            @pl.when(kk > 0)
            def _():
                wait_out(1)

            vpu(1)
            issue_out(c1, 1)
            return 0

        lax.fori_loop(0, npair, body, 0)
        wait_out(0)
        wait_out(1)

    return k(inp, mh, amh, b2a, b2revb)


def kernel(f_atoms, f_bonds, a2b, b2a, b2revb, W_i, W_h, W_o, b_o):
    n_atoms, max_nb = a2b.shape
    fd = f_atoms.shape[1]

    info = plsc.get_sparse_core_info()
    nc, ns = info.num_cores, info.num_subcores
    nw = nc * ns
    apc = 2

    grp = nw * apc * 4
    n_pad = ((n_atoms + grp - 1) // grp) * grp

    b2a = b2a.astype(jnp.int32)
    b2revb = b2revb.astype(jnp.int32)
    a2b_flat = jnp.pad(a2b.astype(jnp.int32),
                       ((0, n_pad - n_atoms), (0, 0))).reshape(-1)

    P = _mm_out1(f_atoms, W_o[:fd], b_o)
    inp_i, mh_i = _mm_first(f_bonds, W_i, W_h)
    amh_i = _gather_sum(mh_i, a2b_flat, n_pad, max_nb, nc, ns, apc)
    msg_i = _gather_msg(inp_i, mh_i, amh_i, b2a, b2revb, nc, ns)
    for _ in range(DEPTH - 2):
        mh_i = _mm_matmul(msg_i, W_h)
        asum_i = _gather_sum(msg_i, a2b_flat, n_pad, max_nb, nc, ns, apc)
        amh_i = _mm_matmul(asum_i, W_h)
        msg_i = _gather_msg(inp_i, mh_i, amh_i, b2a, b2revb, nc, ns)

    a_msg_i = _gather_sum(msg_i, a2b_flat, n_pad, max_nb, nc, ns, apc)
    return _mm_out2(P, a_msg_i[:n_atoms], W_o[fd:])
